# baseline (speedup 1.0000x reference)
import jax
import jax.numpy as jnp
from jax import lax
from jax.experimental import pallas as pl
from jax.experimental.pallas import tpu as pltpu
from jax.experimental.pallas import tpu_sc as plsc

N_NODES = 10000
N_EDGES = 320000
D_FEAT = 128

NUM_CORES = 2
NUM_SUBCORES = 16
NUM_WORKERS = NUM_CORES * NUM_SUBCORES
EDGES_PER_WORKER = N_EDGES // NUM_WORKERS
CHUNK = 80
NUM_CHUNKS = EDGES_PER_WORKER // CHUNK
EDGE_UNROLL = 4


def _dot_chunk(urows, vrows, outbuf, out_off):
    lanes = lax.iota(jnp.int32, 16)
    last_lane = lanes == 15

    def e_body(i, carry):
        for t in range(EDGE_UNROLL):
            e = i * EDGE_UNROLL + t
            acc = None
            for q in range(D_FEAT // 32):
                u = urows[e, pl.ds(32 * q, 32)]
                v = vrows[e, pl.ds(32 * q, 32)]
                p = u * v
                acc = p if acc is None else acc + p
            lo, hi = plsc.unpack(acc, format=plsc.PackFormat.INTERLEAVED)
            cs = plsc.cumsum(lo + hi)
            idx = jnp.full((16,), out_off + e, dtype=jnp.int32)
            plsc.store_scatter(outbuf, [idx], cs, mask=last_lane)
        return carry

    lax.fori_loop(0, CHUNK // EDGE_UNROLL, e_body, 0)


def _sc_kernel(h_hbm, src_hbm, dst_hbm, out_hbm,
               srcbuf, dstbuf, u0, v0, u1, v1, outbuf, h_spmem,
               su0, sv0, su1, sv1):
    sid = lax.axis_index("s")
    wid = sid * NUM_CORES + lax.axis_index("c")
    wbase = wid * EDGES_PER_WORKER
    @pl.when(sid == 0)
    def _():
        pltpu.sync_copy(h_hbm, h_spmem)
    pltpu.sync_copy(src_hbm.at[pl.ds(wbase, EDGES_PER_WORKER)], srcbuf)
    pltpu.sync_copy(dst_hbm.at[pl.ds(wbase, EDGES_PER_WORKER)], dstbuf)
    plsc.subcore_barrier()

    def gather_pair(c, ub, vb, su, sv):
        off = c * CHUNK
        pltpu.async_copy(h_hbm.at[srcbuf.at[pl.ds(off, CHUNK)]], ub, su)
        pltpu.async_copy(h_spmem.at[dstbuf.at[pl.ds(off, CHUNK)]], vb, sv)

    def wait_pair(c, ub, vb, su, sv):
        off = c * CHUNK
        pltpu.make_async_copy(
            h_hbm.at[srcbuf.at[pl.ds(off, CHUNK)]], ub, su).wait()
        pltpu.make_async_copy(
            h_spmem.at[dstbuf.at[pl.ds(off, CHUNK)]], vb, sv).wait()

    gather_pair(0, u0, v0, su0, sv0)

    def body(g, carry):
        c0 = 2 * g
        c1 = c0 + 1
        gather_pair(c1, u1, v1, su1, sv1)
        wait_pair(c0, u0, v0, su0, sv0)
        _dot_chunk(u0, v0, outbuf, c0 * CHUNK)
        gather_pair(c0 + 2, u0, v0, su0, sv0)
        wait_pair(c1, u1, v1, su1, sv1)
        _dot_chunk(u1, v1, outbuf, c1 * CHUNK)
        return carry

    lax.fori_loop(0, (NUM_CHUNKS - 1) // 2, body, 0)
    last = NUM_CHUNKS - 1
    wait_pair(last, u0, v0, su0, sv0)
    _dot_chunk(u0, v0, outbuf, last * CHUNK)

    pltpu.sync_copy(outbuf, out_hbm.at[pl.ds(wbase, EDGES_PER_WORKER)])


@jax.jit
def kernel(h, edge_index):
    h_bf = h.astype(jnp.bfloat16)
    src = edge_index[0]
    dst = edge_index[1]
    mesh = plsc.VectorSubcoreMesh(core_axis_name="c", subcore_axis_name="s")
    k = pl.kernel(
        _sc_kernel,
        out_type=jax.ShapeDtypeStruct((N_EDGES,), jnp.float32),
        mesh=mesh,
        compiler_params=pltpu.CompilerParams(
            use_tc_tiling_on_sc=False, needs_layout_passes=False),
        scratch_types=[
            pltpu.VMEM((EDGES_PER_WORKER,), jnp.int32),
            pltpu.VMEM((EDGES_PER_WORKER,), jnp.int32),
            pltpu.VMEM((CHUNK, D_FEAT), jnp.bfloat16),
            pltpu.VMEM((CHUNK, D_FEAT), jnp.bfloat16),
            pltpu.VMEM((CHUNK, D_FEAT), jnp.bfloat16),
            pltpu.VMEM((CHUNK, D_FEAT), jnp.bfloat16),
            pltpu.VMEM((EDGES_PER_WORKER,), jnp.float32),
            pltpu.VMEM_SHARED((N_NODES, D_FEAT), jnp.bfloat16),
            pltpu.SemaphoreType.DMA,
            pltpu.SemaphoreType.DMA,
            pltpu.SemaphoreType.DMA,
            pltpu.SemaphoreType.DMA,
        ],
    )
    return k(h_bf, src, dst)


# speedup vs baseline: 1.4372x; 1.4372x over previous
import jax
import jax.numpy as jnp
from jax import lax
from jax.experimental import pallas as pl
from jax.experimental.pallas import tpu as pltpu
from jax.experimental.pallas import tpu_sc as plsc

N_NODES = 10000
N_EDGES = 320000
D_FEAT = 128

NUM_CORES = 2
NUM_SUBCORES = 16
NUM_WORKERS = NUM_CORES * NUM_SUBCORES
EDGES_PER_WORKER = N_EDGES // NUM_WORKERS
CHUNK = 80
NUM_CHUNKS = EDGES_PER_WORKER // CHUNK
BLOCKS_PER_CHUNK = CHUNK // 16
K_UNROLL = 8
NBUF = 4
MAIN_CHUNKS = (NUM_CHUNKS // NBUF - 1) * NBUF


def _dot_chunk(urows, vrows, outbuf, out_off):
    lanes = lax.iota(jnp.int32, 16)
    for b in range(BLOCKS_PER_CHUNK):
        rows = lanes + (16 * b)

        def k_body(i, acc):
            for u in range(K_UNROLL):
                col = (lanes + (i * K_UNROLL + u)) & (D_FEAT - 1)
                uv = plsc.load_gather(urows, [rows, col])
                vv = plsc.load_gather(vrows, [rows, col])
                acc = acc + uv * vv
            return acc

        acc = lax.fori_loop(0, D_FEAT // K_UNROLL, k_body,
                            jnp.zeros((16,), jnp.float32))
        outbuf[pl.ds(out_off + 16 * b, 16)] = acc


def _sc_kernel(h_hbm, src_hbm, dst_hbm, out_hbm,
               srcbuf, dstbuf, ubufs, vbufs, outbuf, usems, vsems):
    wid = lax.axis_index("s") * NUM_CORES + lax.axis_index("c")
    wbase = wid * EDGES_PER_WORKER
    pltpu.sync_copy(src_hbm.at[pl.ds(wbase, EDGES_PER_WORKER)], srcbuf)
    pltpu.sync_copy(dst_hbm.at[pl.ds(wbase, EDGES_PER_WORKER)], dstbuf)

    def gather_pair(c, s):
        off = c * CHUNK
        pltpu.async_copy(
            h_hbm.at[srcbuf.at[pl.ds(off, CHUNK)]], ubufs[s], usems[s])
        pltpu.async_copy(
            h_hbm.at[dstbuf.at[pl.ds(off, CHUNK)]], vbufs[s], vsems[s])

    def wait_pair(c, s):
        off = c * CHUNK
        pltpu.make_async_copy(
            h_hbm.at[srcbuf.at[pl.ds(off, CHUNK)]], ubufs[s], usems[s]).wait()
        pltpu.make_async_copy(
            h_hbm.at[dstbuf.at[pl.ds(off, CHUNK)]], vbufs[s], vsems[s]).wait()

    for s in range(NBUF - 1):
        gather_pair(s, s)

    def body(g, carry):
        c_base = NBUF * g
        for s in range(NBUF):
            c = c_base + s
            wait_pair(c, s)
            gather_pair(c + NBUF - 1, (s + NBUF - 1) % NBUF)
            _dot_chunk(ubufs[s], vbufs[s], outbuf, c * CHUNK)
        return carry

    lax.fori_loop(0, MAIN_CHUNKS // NBUF, body, 0)
    for c in range(MAIN_CHUNKS, NUM_CHUNKS):
        s = c % NBUF
        wait_pair(c, s)
        if c + NBUF - 1 < NUM_CHUNKS:
            gather_pair(c + NBUF - 1, (s + NBUF - 1) % NBUF)
        _dot_chunk(ubufs[s], vbufs[s], outbuf, c * CHUNK)

    pltpu.sync_copy(outbuf, out_hbm.at[pl.ds(wbase, EDGES_PER_WORKER)])


@jax.jit
def kernel(h, edge_index):
    src = edge_index[0]
    dst = edge_index[1]
    mesh = plsc.VectorSubcoreMesh(core_axis_name="c", subcore_axis_name="s")
    k = pl.kernel(
        _sc_kernel,
        out_type=jax.ShapeDtypeStruct((N_EDGES,), jnp.float32),
        mesh=mesh,
        compiler_params=pltpu.CompilerParams(
            use_tc_tiling_on_sc=False, needs_layout_passes=False),
        scratch_types=[
            pltpu.VMEM((EDGES_PER_WORKER,), jnp.int32),
            pltpu.VMEM((EDGES_PER_WORKER,), jnp.int32),
            [pltpu.VMEM((CHUNK, D_FEAT), jnp.float32) for _ in range(NBUF)],
            [pltpu.VMEM((CHUNK, D_FEAT), jnp.float32) for _ in range(NBUF)],
            pltpu.VMEM((EDGES_PER_WORKER,), jnp.float32),
            [pltpu.SemaphoreType.DMA for _ in range(NBUF)],
            [pltpu.SemaphoreType.DMA for _ in range(NBUF)],
        ],
    )
    return k(h, src, dst)
